# baseline (speedup 1.0000x reference)
import functools

import jax
import jax.numpy as jnp
from jax.experimental import pallas as pl
from jax.experimental.pallas import tpu as pltpu

E = 8
TOP_K = 2
TM = 256


def _moe_kernel(mg_ref, dom_ref, x_ref, gw_ref, gu_ref, dn_ref, out_ref, *, num_groups):
    g = pl.program_id(0)
    t = pl.program_id(1)

    xt = x_ref[...]

    logits = jax.lax.dot_general(
        xt, gw_ref[...], (((1,), (1,)), ((), ())),
        preferred_element_type=jnp.float32)
    m = jnp.max(logits, axis=1, keepdims=True)
    ex = jnp.exp(logits - m)
    probs = ex / jnp.sum(ex, axis=1, keepdims=True)

    i1 = jnp.argmax(probs, axis=1)
    v1 = jnp.max(probs, axis=1)
    iota = jax.lax.broadcasted_iota(jnp.int32, probs.shape, 1)
    masked = jnp.where(iota == i1[:, None], -jnp.inf, probs)
    i2 = jnp.argmax(masked, axis=1)
    v2 = jnp.max(masked, axis=1)
    denom = v1 + v2

    wg = jnp.zeros_like(v1)
    for e in range(E):
        in_g = mg_ref[e] == g
        sel = jnp.where(i1 == e, v1, 0.0) + jnp.where(i2 == e, v2, 0.0)
        wg = wg + jnp.where(in_g, sel, 0.0)
    wg = wg / denom

    xt_bf = xt.astype(jnp.bfloat16)
    gu = jax.lax.dot_general(
        xt_bf, gu_ref[0], (((1,), (1,)), ((), ())),
        preferred_element_type=jnp.float32)
    dff = gu.shape[1] // 2
    gate_h = gu[:, :dff]
    up_h = gu[:, dff:]
    h = gate_h * jax.lax.logistic(gate_h) * up_h
    out = jax.lax.dot_general(
        h.astype(jnp.bfloat16), dn_ref[0], (((1,), (1,)), ((), ())),
        preferred_element_type=jnp.float32)
    out = out * wg[:, None]

    sl = pl.ds(t * TM, TM)

    @pl.when(g == 0)
    def _init():
        out_ref[sl, :] = out

    @pl.when(g != 0)
    def _acc():
        out_ref[sl, :] = out_ref[sl, :] + out


def kernel(hidden_states, gate_weight, gate_up_proj, down_proj, merge_groups, dominant_experts):
    b, s, d = hidden_states.shape
    x = hidden_states.reshape(s, d)
    gate_up_proj = gate_up_proj.astype(jnp.bfloat16)
    down_proj = down_proj.astype(jnp.bfloat16)
    num_groups = dominant_experts.shape[0]
    two_dff = gate_up_proj.shape[1]
    n_t = s // TM

    grid_spec = pltpu.PrefetchScalarGridSpec(
        num_scalar_prefetch=2,
        grid=(num_groups, n_t),
        in_specs=[
            pl.BlockSpec((TM, d), lambda g, t, mg, dom: (t, 0)),
            pl.BlockSpec((E, d), lambda g, t, mg, dom: (0, 0)),
            pl.BlockSpec((1, two_dff, d), lambda g, t, mg, dom: (dom[g], 0, 0)),
            pl.BlockSpec((1, d, down_proj.shape[2]), lambda g, t, mg, dom: (dom[g], 0, 0)),
        ],
        out_specs=pl.BlockSpec((s, d), lambda g, t, mg, dom: (0, 0)),
    )

    out = pl.pallas_call(
        functools.partial(_moe_kernel, num_groups=num_groups),
        grid_spec=grid_spec,
        out_shape=jax.ShapeDtypeStruct((s, d), x.dtype),
        compiler_params=pltpu.CompilerParams(
            dimension_semantics=("arbitrary", "arbitrary"),
        ),
    )(merge_groups, dominant_experts, x, gate_weight, gate_up_proj, down_proj)
    return out.reshape(b, s, d)


# speedup vs baseline: 1.5628x; 1.5628x over previous
import functools

import jax
import jax.numpy as jnp
from jax.experimental import pallas as pl
from jax.experimental.pallas import tpu as pltpu

E = 8
TOP_K = 2
TM = 512


def _moe_kernel(mg_ref, dom_ref, x_ref, gw_ref, gu_ref, dn_ref, out_ref, *, num_groups):
    g = pl.program_id(0)
    t = pl.program_id(1)

    xt = x_ref[...]

    logits = jax.lax.dot_general(
        xt, gw_ref[...], (((1,), (1,)), ((), ())),
        preferred_element_type=jnp.float32)
    m = jnp.max(logits, axis=1, keepdims=True)
    ex = jnp.exp(logits - m)
    probs = ex / jnp.sum(ex, axis=1, keepdims=True)

    i1 = jnp.argmax(probs, axis=1)
    v1 = jnp.max(probs, axis=1)
    iota = jax.lax.broadcasted_iota(jnp.int32, probs.shape, 1)
    masked = jnp.where(iota == i1[:, None], -jnp.inf, probs)
    i2 = jnp.argmax(masked, axis=1)
    v2 = jnp.max(masked, axis=1)
    denom = v1 + v2

    wg = jnp.zeros_like(v1)
    for e in range(E):
        in_g = mg_ref[e] == g
        sel = jnp.where(i1 == e, v1, 0.0) + jnp.where(i2 == e, v2, 0.0)
        wg = wg + jnp.where(in_g, sel, 0.0)
    wg = wg / denom

    gu = jax.lax.dot_general(
        xt, gu_ref[0], (((1,), (1,)), ((), ())),
        preferred_element_type=jnp.float32)
    dff = gu.shape[1] // 2
    gate_h = gu[:, :dff]
    up_h = gu[:, dff:]
    h = gate_h * jax.lax.logistic(gate_h) * up_h
    out = jax.lax.dot_general(
        h, dn_ref[0], (((1,), (1,)), ((), ())),
        preferred_element_type=jnp.float32)
    out = out * wg[:, None]

    sl = pl.ds(t * TM, TM)

    @pl.when(g == 0)
    def _init():
        out_ref[sl, :] = out

    @pl.when(g != 0)
    def _acc():
        out_ref[sl, :] = out_ref[sl, :] + out


def kernel(hidden_states, gate_weight, gate_up_proj, down_proj, merge_groups, dominant_experts):
    b, s, d = hidden_states.shape
    x = hidden_states.reshape(s, d)
    num_groups = dominant_experts.shape[0]
    two_dff = gate_up_proj.shape[1]
    n_t = s // TM

    grid_spec = pltpu.PrefetchScalarGridSpec(
        num_scalar_prefetch=2,
        grid=(num_groups, n_t),
        in_specs=[
            pl.BlockSpec((TM, d), lambda g, t, mg, dom: (t, 0)),
            pl.BlockSpec((E, d), lambda g, t, mg, dom: (0, 0)),
            pl.BlockSpec((1, two_dff, d), lambda g, t, mg, dom: (dom[g], 0, 0)),
            pl.BlockSpec((1, d, down_proj.shape[2]), lambda g, t, mg, dom: (dom[g], 0, 0)),
        ],
        out_specs=pl.BlockSpec((s, d), lambda g, t, mg, dom: (0, 0)),
    )

    out = pl.pallas_call(
        functools.partial(_moe_kernel, num_groups=num_groups),
        grid_spec=grid_spec,
        out_shape=jax.ShapeDtypeStruct((s, d), x.dtype),
        compiler_params=pltpu.CompilerParams(
            dimension_semantics=("arbitrary", "arbitrary"),
        ),
    )(merge_groups, dominant_experts, x, gate_weight, gate_up_proj, down_proj)
    return out.reshape(b, s, d)


# speedup vs baseline: 1.6505x; 1.0562x over previous
import functools

import jax
import jax.numpy as jnp
from jax.experimental import pallas as pl
from jax.experimental.pallas import tpu as pltpu

E = 8
TOP_K = 2
TM = 1024


def _moe_kernel(mg_ref, dom_ref, x_ref, gw_ref, gu_ref, dn_ref, out_ref, *, num_groups):
    g = pl.program_id(0)
    t = pl.program_id(1)

    xt = x_ref[...]

    logits = jax.lax.dot_general(
        xt, gw_ref[...], (((1,), (1,)), ((), ())),
        preferred_element_type=jnp.float32)
    m = jnp.max(logits, axis=1, keepdims=True)
    ex = jnp.exp(logits - m)
    probs = ex / jnp.sum(ex, axis=1, keepdims=True)

    i1 = jnp.argmax(probs, axis=1)
    v1 = jnp.max(probs, axis=1)
    iota = jax.lax.broadcasted_iota(jnp.int32, probs.shape, 1)
    masked = jnp.where(iota == i1[:, None], -jnp.inf, probs)
    i2 = jnp.argmax(masked, axis=1)
    v2 = jnp.max(masked, axis=1)
    denom = v1 + v2

    wg = jnp.zeros_like(v1)
    for e in range(E):
        in_g = mg_ref[e] == g
        sel = jnp.where(i1 == e, v1, 0.0) + jnp.where(i2 == e, v2, 0.0)
        wg = wg + jnp.where(in_g, sel, 0.0)
    wg = wg / denom

    gu = jax.lax.dot_general(
        xt, gu_ref[0], (((1,), (1,)), ((), ())),
        preferred_element_type=jnp.float32)
    dff = gu.shape[1] // 2
    gate_h = gu[:, :dff]
    up_h = gu[:, dff:]
    h = gate_h * jax.lax.logistic(gate_h) * up_h
    out = jax.lax.dot_general(
        h, dn_ref[0], (((1,), (1,)), ((), ())),
        preferred_element_type=jnp.float32)
    out = out * wg[:, None]

    sl = pl.ds(t * TM, TM)

    @pl.when(g == 0)
    def _init():
        out_ref[sl, :] = out

    @pl.when(g != 0)
    def _acc():
        out_ref[sl, :] = out_ref[sl, :] + out


def kernel(hidden_states, gate_weight, gate_up_proj, down_proj, merge_groups, dominant_experts):
    b, s, d = hidden_states.shape
    x = hidden_states.reshape(s, d)
    num_groups = dominant_experts.shape[0]
    two_dff = gate_up_proj.shape[1]
    n_t = s // TM

    grid_spec = pltpu.PrefetchScalarGridSpec(
        num_scalar_prefetch=2,
        grid=(num_groups, n_t),
        in_specs=[
            pl.BlockSpec((TM, d), lambda g, t, mg, dom: (t, 0)),
            pl.BlockSpec((E, d), lambda g, t, mg, dom: (0, 0)),
            pl.BlockSpec((1, two_dff, d), lambda g, t, mg, dom: (dom[g], 0, 0)),
            pl.BlockSpec((1, d, down_proj.shape[2]), lambda g, t, mg, dom: (dom[g], 0, 0)),
        ],
        out_specs=pl.BlockSpec((s, d), lambda g, t, mg, dom: (0, 0)),
    )

    out = pl.pallas_call(
        functools.partial(_moe_kernel, num_groups=num_groups),
        grid_spec=grid_spec,
        out_shape=jax.ShapeDtypeStruct((s, d), x.dtype),
        compiler_params=pltpu.CompilerParams(
            dimension_semantics=("arbitrary", "arbitrary"),
        ),
    )(merge_groups, dominant_experts, x, gate_weight, gate_up_proj, down_proj)
    return out.reshape(b, s, d)


# speedup vs baseline: 1.7811x; 1.0791x over previous
import functools

import jax
import jax.numpy as jnp
from jax.experimental import pallas as pl
from jax.experimental.pallas import tpu as pltpu

E = 8
TOP_K = 2
TM = 1024


def _moe_kernel(mg_ref, dom_ref, x_ref, gw_ref, gu_ref, dn_ref, out_ref, wg_ref,
                *, num_groups):
    g = pl.program_id(0)
    t = pl.program_id(1)
    sl = pl.ds(t * TM, TM)

    xt = x_ref[...]

    @pl.when(g == 0)
    def _router():
        logits = jax.lax.dot_general(
            xt, gw_ref[...], (((1,), (1,)), ((), ())),
            preferred_element_type=jnp.float32)
        m = jnp.max(logits, axis=1, keepdims=True)
        ex = jnp.exp(logits - m)
        probs = ex / jnp.sum(ex, axis=1, keepdims=True)

        i1 = jnp.argmax(probs, axis=1)
        v1 = jnp.max(probs, axis=1)
        iota = jax.lax.broadcasted_iota(jnp.int32, probs.shape, 1)
        masked = jnp.where(iota == i1[:, None], -jnp.inf, probs)
        i2 = jnp.argmax(masked, axis=1)
        v2 = jnp.max(masked, axis=1)
        denom = v1 + v2

        for gg in range(num_groups):
            wg = jnp.zeros_like(v1)
            for e in range(E):
                in_g = mg_ref[e] == gg
                sel = jnp.where(i1 == e, v1, 0.0) + jnp.where(i2 == e, v2, 0.0)
                wg = wg + jnp.where(in_g, sel, 0.0)
            wg_ref[gg, sl] = wg / denom

    gu = jax.lax.dot_general(
        xt, gu_ref[0], (((1,), (1,)), ((), ())),
        preferred_element_type=jnp.float32)
    dff = gu.shape[1] // 2
    gate_h = gu[:, :dff]
    up_h = gu[:, dff:]
    h = gate_h * jax.lax.logistic(gate_h) * up_h

    wg_row = wg_ref[g, sl]
    h = h * wg_row[:, None]

    out = jax.lax.dot_general(
        h, dn_ref[0], (((1,), (1,)), ((), ())),
        preferred_element_type=jnp.float32)

    @pl.when(g == 0)
    def _init():
        out_ref[sl, :] = out

    @pl.when(g != 0)
    def _acc():
        out_ref[sl, :] = out_ref[sl, :] + out


def kernel(hidden_states, gate_weight, gate_up_proj, down_proj, merge_groups, dominant_experts):
    b, s, d = hidden_states.shape
    x = hidden_states.reshape(s, d)
    num_groups = dominant_experts.shape[0]
    two_dff = gate_up_proj.shape[1]
    n_t = s // TM

    grid_spec = pltpu.PrefetchScalarGridSpec(
        num_scalar_prefetch=2,
        grid=(num_groups, n_t),
        in_specs=[
            pl.BlockSpec((TM, d), lambda g, t, mg, dom: (t, 0)),
            pl.BlockSpec((E, d), lambda g, t, mg, dom: (0, 0)),
            pl.BlockSpec((1, two_dff, d), lambda g, t, mg, dom: (dom[g], 0, 0)),
            pl.BlockSpec((1, d, down_proj.shape[2]), lambda g, t, mg, dom: (dom[g], 0, 0)),
        ],
        out_specs=pl.BlockSpec((s, d), lambda g, t, mg, dom: (0, 0)),
        scratch_shapes=[pltpu.VMEM((num_groups, s), jnp.float32)],
    )

    out = pl.pallas_call(
        functools.partial(_moe_kernel, num_groups=num_groups),
        grid_spec=grid_spec,
        out_shape=jax.ShapeDtypeStruct((s, d), x.dtype),
        compiler_params=pltpu.CompilerParams(
            dimension_semantics=("arbitrary", "arbitrary"),
        ),
    )(merge_groups, dominant_experts, x, gate_weight, gate_up_proj, down_proj)
    return out.reshape(b, s, d)


# speedup vs baseline: 1.7840x; 1.0016x over previous
import functools

import jax
import jax.numpy as jnp
from jax.experimental import pallas as pl
from jax.experimental.pallas import tpu as pltpu

E = 8
TOP_K = 2
TM = 1024


def _moe_kernel(mg_ref, dom_ref, x_ref, gw_ref, gu_ref, dn_ref, out_ref, wg_ref,
                *, num_groups):
    g = pl.program_id(0)
    t = pl.program_id(1)
    sl = pl.ds(t * TM, TM)

    xt = x_ref[...]

    @pl.when(g == 0)
    def _router():
        logits = jax.lax.dot_general(
            xt, gw_ref[...], (((1,), (1,)), ((), ())),
            preferred_element_type=jnp.float32)
        m = jnp.max(logits, axis=1, keepdims=True)
        ex = jnp.exp(logits - m)
        probs = ex / jnp.sum(ex, axis=1, keepdims=True)

        i1 = jnp.argmax(probs, axis=1)
        v1 = jnp.max(probs, axis=1)
        iota = jax.lax.broadcasted_iota(jnp.int32, probs.shape, 1)
        masked = jnp.where(iota == i1[:, None], -jnp.inf, probs)
        i2 = jnp.argmax(masked, axis=1)
        v2 = jnp.max(masked, axis=1)
        denom = v1 + v2

        for gg in range(num_groups):
            wg = jnp.zeros_like(v1)
            for e in range(E):
                in_g = mg_ref[e] == gg
                sel = jnp.where(i1 == e, v1, 0.0) + jnp.where(i2 == e, v2, 0.0)
                wg = wg + jnp.where(in_g, sel, 0.0)
            wg_ref[gg, sl] = wg / denom

    gu = jax.lax.dot_general(
        xt, gu_ref[0], (((1,), (1,)), ((), ())),
        preferred_element_type=jnp.float32)
    dff = gu.shape[1] // 2
    gate_h = gu[:, :dff]
    up_h = gu[:, dff:]
    h = gate_h * jax.lax.logistic(gate_h) * up_h

    out = jax.lax.dot_general(
        h, dn_ref[0], (((1,), (1,)), ((), ())),
        preferred_element_type=jnp.float32)
    wg_row = wg_ref[g, sl]
    out = out * wg_row[:, None]

    @pl.when(g == 0)
    def _init():
        out_ref[sl, :] = out

    @pl.when(g != 0)
    def _acc():
        out_ref[sl, :] = out_ref[sl, :] + out


def kernel(hidden_states, gate_weight, gate_up_proj, down_proj, merge_groups, dominant_experts):
    b, s, d = hidden_states.shape
    x = hidden_states.reshape(s, d)
    num_groups = dominant_experts.shape[0]
    two_dff = gate_up_proj.shape[1]
    n_t = s // TM

    grid_spec = pltpu.PrefetchScalarGridSpec(
        num_scalar_prefetch=2,
        grid=(num_groups, n_t),
        in_specs=[
            pl.BlockSpec((TM, d), lambda g, t, mg, dom: (t, 0)),
            pl.BlockSpec((E, d), lambda g, t, mg, dom: (0, 0)),
            pl.BlockSpec((1, two_dff, d), lambda g, t, mg, dom: (dom[g], 0, 0)),
            pl.BlockSpec((1, d, down_proj.shape[2]), lambda g, t, mg, dom: (dom[g], 0, 0)),
        ],
        out_specs=pl.BlockSpec((s, d), lambda g, t, mg, dom: (0, 0)),
        scratch_shapes=[pltpu.VMEM((num_groups, s), jnp.float32)],
    )

    out = pl.pallas_call(
        functools.partial(_moe_kernel, num_groups=num_groups),
        grid_spec=grid_spec,
        out_shape=jax.ShapeDtypeStruct((s, d), x.dtype),
        compiler_params=pltpu.CompilerParams(
            dimension_semantics=("arbitrary", "arbitrary"),
        ),
    )(merge_groups, dominant_experts, x, gate_weight, gate_up_proj, down_proj)
    return out.reshape(b, s, d)
